# baseline (speedup 1.0000x reference)
import jax
import jax.numpy as jnp
from jax import lax
from jax.experimental import pallas as pl
from jax.experimental.pallas import tpu as pltpu
from jax.experimental.pallas import tpu_sc as plsc

FEAT = 128
NUM_K = 1024
POOL_PER = 256
TOTAL = NUM_K * POOL_PER
M = 16384

NUM_TILES = 32
CPT = NUM_K // NUM_TILES
CAP = 40
IDXBUF = 48
SCOL_LEN = 64
STARTS_LEN = 48
STARTS_PAD = 1088
UPD_PAD = M + IDXBUF


def _sc_body(act_hbm, rows_hbm, scol_hbm, starts_hbm, pool_hbm, out_hbm,
             starts_v, rowid_v, scol_v, rows_v, blocks_v,
             lsem, ssem, isem, gsem):
    wid = lax.axis_index("c") * 16 + lax.axis_index("s")
    c0 = wid * CPT

    pltpu.sync_copy(starts_hbm.at[pl.ds(c0, STARTS_LEN)], starts_v)

    def cluster_start(j):
        return starts_v[pl.ds(j, 16)][0]

    def col_window(j):
        return pl.ds((c0 + j) * POOL_PER, POOL_PER)

    def start_idx(j, b):
        a = (cluster_start(j) // 8) * 8
        pltpu.async_copy(rows_hbm.at[pl.ds(a, IDXBUF)], rowid_v.at[b],
                         isem.at[b])
        pltpu.async_copy(scol_hbm.at[pl.ds(a, IDXBUF)],
                         scol_v.at[b].at[pl.ds(0, IDXBUF)], isem.at[b])

    def wait_idx(b):
        pltpu.make_async_copy(rows_hbm.at[pl.ds(0, IDXBUF)], rowid_v.at[b],
                              isem.at[b]).wait()
        pltpu.make_async_copy(scol_hbm.at[pl.ds(0, IDXBUF)],
                              scol_v.at[b].at[pl.ds(0, IDXBUF)],
                              isem.at[b]).wait()

    def issue_gather(b):
        pltpu.async_copy(act_hbm.at[rowid_v.at[b]], rows_v.at[b], gsem.at[b])

    def wait_gather(b):
        pltpu.make_async_copy(act_hbm.at[rowid_v.at[b]], rows_v.at[b],
                              gsem.at[b]).wait()

    def start_load(j, b):
        pltpu.async_copy(pool_hbm.at[:, col_window(j)], blocks_v.at[b],
                         lsem.at[b])

    def wait_load(j, b):
        pltpu.make_async_copy(pool_hbm.at[:, col_window(j)], blocks_v.at[b],
                              lsem.at[b]).wait()

    def start_store(j, b):
        pltpu.async_copy(blocks_v.at[b], out_hbm.at[:, col_window(j)],
                         ssem.at[b])

    def wait_store(b):
        pltpu.make_async_copy(blocks_v.at[b], out_hbm.at[:, col_window(0)],
                              ssem.at[b]).wait()

    def apply_range(lo, hi, a, b, col0):
        def apply(p, _):
            q = p - a
            o = scol_v[b, pl.ds(q, 16)][0] - col0
            cidx = jnp.full((16,), o, dtype=jnp.int32)
            for fv in range(FEAT // 16):
                vals = rows_v[b, q, pl.ds(fv * 16, 16)]
                ridx = lax.iota(jnp.int32, 16) + fv * 16
                plsc.store_scatter(blocks_v.at[b], [ridx, cidx], vals)
            return 0

        lax.fori_loop(lo, hi, apply, 0)

    def apply_cluster(j, b):
        col0 = (c0 + j) * POOL_PER
        sv = starts_v[pl.ds(j, 16)]
        s = sv[0]
        e = sv[1]
        apply_range(s, jnp.minimum(s + CAP, e), (s // 8) * 8, b, col0)
        nch = (e - s + (CAP - 1)) // CAP

        def rare(k, _):
            base = s + k * CAP
            a = (base // 8) * 8
            pltpu.sync_copy(rows_hbm.at[pl.ds(a, IDXBUF)], rowid_v.at[b])
            pltpu.sync_copy(scol_hbm.at[pl.ds(a, IDXBUF)],
                            scol_v.at[b].at[pl.ds(0, IDXBUF)])
            pltpu.async_copy(act_hbm.at[rowid_v.at[b]], rows_v.at[b],
                             gsem.at[b]).wait()
            apply_range(base, jnp.minimum(base + CAP, e), a, b, col0)
            return 0

        lax.fori_loop(1, nch, rare, 0)

    def step(j, r):
        nb = (r + 1) % 3
        pb = (r + 2) % 3

        @pl.when(j >= 2)
        def _():
            wait_store(nb)

        start_load(j + 1, nb)
        start_idx(j + 2, pb)
        wait_idx(nb)
        issue_gather(nb)
        wait_load(j, r)
        wait_gather(r)
        apply_cluster(j, r)
        start_store(j, r)

    start_idx(0, 0)
    wait_idx(0)
    issue_gather(0)
    start_idx(1, 1)
    start_load(0, 0)

    def loop(i, _):
        for r in range(3):
            step(3 * i + r, r)
        return 0

    lax.fori_loop(0, 10, loop, 0)

    wait_store(1)
    start_load(31, 1)
    wait_idx(1)
    issue_gather(1)
    wait_load(30, 0)
    wait_gather(0)
    apply_cluster(30, 0)
    start_store(30, 0)

    wait_load(31, 1)
    wait_gather(1)
    apply_cluster(31, 1)
    start_store(31, 1)

    wait_store(0)
    wait_store(1)
    wait_store(2)


def kernel(activation, cluster_num, rand_offsets, concept_pool):
    idx = (cluster_num.astype(jnp.int32) * POOL_PER
           + rand_offsets.astype(jnp.int32))
    starts_p = jnp.zeros((STARTS_PAD,), jnp.int32)
    rows_p = jnp.zeros((UPD_PAD,), jnp.int32) + idx[0] * 0
    scol_p = jnp.zeros((UPD_PAD,), jnp.int32)

    mesh = plsc.VectorSubcoreMesh(core_axis_name="c", subcore_axis_name="s",
                                  num_cores=2, num_subcores=16)
    run = pl.kernel(
        _sc_body,
        out_type=jax.ShapeDtypeStruct((FEAT, TOTAL), jnp.float32),
        mesh=mesh,
        scratch_types=[
            pltpu.VMEM((STARTS_LEN,), jnp.int32),
            pltpu.VMEM((3, IDXBUF), jnp.int32),
            pltpu.VMEM((3, SCOL_LEN), jnp.int32),
            pltpu.VMEM((3, IDXBUF, FEAT), jnp.float32),
            pltpu.VMEM((3, FEAT, POOL_PER), jnp.float32),
            pltpu.SemaphoreType.DMA((3,)),
            pltpu.SemaphoreType.DMA((3,)),
            pltpu.SemaphoreType.DMA((3,)),
            pltpu.SemaphoreType.DMA((3,)),
        ],
        compiler_params=pltpu.CompilerParams(use_tc_tiling_on_sc=True,
                                             needs_layout_passes=False),
    )
    return run(activation, rows_p, scol_p, starts_p, concept_pool)


# speedup vs baseline: 15.6935x; 15.6935x over previous
import jax
import jax.numpy as jnp
from jax import lax
from jax.experimental import pallas as pl
from jax.experimental.pallas import tpu as pltpu
from jax.experimental.pallas import tpu_sc as plsc

FEAT = 128
NUM_K = 1024
POOL_PER = 256
TOTAL = NUM_K * POOL_PER
M = 16384

NUM_TILES = 32
CPT = NUM_K // NUM_TILES
CAP = 40
IDXBUF = 48
SCOL_LEN = 64
STARTS_LEN = 48
STARTS_PAD = 1088
UPD_PAD = M + IDXBUF


def _sc_body(act_hbm, rows_hbm, scol_hbm, starts_hbm, pool_hbm, out_hbm,
             starts_v, rowid_v, scol_v, rows_v, blocks_v,
             lsem, ssem, isem, gsem):
    wid = lax.axis_index("c") * 16 + lax.axis_index("s")
    c0 = wid * CPT

    pltpu.sync_copy(starts_hbm.at[pl.ds(c0, STARTS_LEN)], starts_v)

    def cluster_start(j):
        return starts_v[pl.ds(j, 16)][0]

    def col_window(j):
        return pl.ds((c0 + j) * POOL_PER, POOL_PER)

    def start_idx(j, b):
        a = (cluster_start(j) // 8) * 8
        pltpu.async_copy(rows_hbm.at[pl.ds(a, IDXBUF)], rowid_v.at[b],
                         isem.at[b])
        pltpu.async_copy(scol_hbm.at[pl.ds(a, IDXBUF)],
                         scol_v.at[b].at[pl.ds(0, IDXBUF)], isem.at[b])

    def wait_idx(b):
        pltpu.make_async_copy(rows_hbm.at[pl.ds(0, IDXBUF)], rowid_v.at[b],
                              isem.at[b]).wait()
        pltpu.make_async_copy(scol_hbm.at[pl.ds(0, IDXBUF)],
                              scol_v.at[b].at[pl.ds(0, IDXBUF)],
                              isem.at[b]).wait()

    def issue_gather(b):
        pltpu.async_copy(act_hbm.at[rowid_v.at[b]], rows_v.at[b], gsem.at[b])

    def wait_gather(b):
        pltpu.make_async_copy(act_hbm.at[rowid_v.at[b]], rows_v.at[b],
                              gsem.at[b]).wait()

    def start_load(j, b):
        pltpu.async_copy(pool_hbm.at[:, col_window(j)], blocks_v.at[b],
                         lsem.at[b])

    def wait_load(j, b):
        pltpu.make_async_copy(pool_hbm.at[:, col_window(j)], blocks_v.at[b],
                              lsem.at[b]).wait()

    def start_store(j, b):
        pltpu.async_copy(blocks_v.at[b], out_hbm.at[:, col_window(j)],
                         ssem.at[b])

    def wait_store(b):
        pltpu.make_async_copy(blocks_v.at[b], out_hbm.at[:, col_window(0)],
                              ssem.at[b]).wait()

    def apply_range(lo, hi, a, b, col0):
        def apply(p, _):
            q = p - a
            o = scol_v[b, pl.ds(q, 16)][0] - col0
            cidx = jnp.full((16,), o, dtype=jnp.int32)
            for fv in range(FEAT // 16):
                vals = rows_v[b, q, pl.ds(fv * 16, 16)]
                ridx = lax.iota(jnp.int32, 16) + fv * 16
                plsc.store_scatter(blocks_v.at[b], [ridx, cidx], vals)
            return 0

        lax.fori_loop(lo, hi, apply, 0)

    def apply_cluster(j, b):
        col0 = (c0 + j) * POOL_PER
        sv = starts_v[pl.ds(j, 16)]
        s = sv[0]
        e = sv[1]
        apply_range(s, jnp.minimum(s + CAP, e), (s // 8) * 8, b, col0)
        nch = (e - s + (CAP - 1)) // CAP

        def rare(k, _):
            base = s + k * CAP
            a = (base // 8) * 8
            pltpu.sync_copy(rows_hbm.at[pl.ds(a, IDXBUF)], rowid_v.at[b])
            pltpu.sync_copy(scol_hbm.at[pl.ds(a, IDXBUF)],
                            scol_v.at[b].at[pl.ds(0, IDXBUF)])
            pltpu.async_copy(act_hbm.at[rowid_v.at[b]], rows_v.at[b],
                             gsem.at[b]).wait()
            apply_range(base, jnp.minimum(base + CAP, e), a, b, col0)
            return 0

        lax.fori_loop(1, nch, rare, 0)

    def step(j, r):
        nb = (r + 1) % 3
        pb = (r + 2) % 3

        @pl.when(j >= 2)
        def _():
            wait_store(nb)

        start_load(j + 1, nb)
        start_idx(j + 2, pb)
        wait_idx(nb)
        issue_gather(nb)
        wait_load(j, r)
        wait_gather(r)
        apply_cluster(j, r)
        start_store(j, r)

    start_idx(0, 0)
    wait_idx(0)
    issue_gather(0)
    start_idx(1, 1)
    start_load(0, 0)

    def loop(i, _):
        for r in range(3):
            step(3 * i + r, r)
        return 0

    lax.fori_loop(0, 10, loop, 0)

    wait_store(1)
    start_load(31, 1)
    wait_idx(1)
    issue_gather(1)
    wait_load(30, 0)
    wait_gather(0)
    apply_cluster(30, 0)
    start_store(30, 0)

    wait_load(31, 1)
    wait_gather(1)
    apply_cluster(31, 1)
    start_store(31, 1)

    wait_store(0)
    wait_store(1)
    wait_store(2)


def kernel(activation, cluster_num, rand_offsets, concept_pool):
    idx = (cluster_num.astype(jnp.int32) * POOL_PER
           + rand_offsets.astype(jnp.int32))
    scol, order = lax.sort_key_val(idx, jnp.arange(M, dtype=jnp.int32),
                                   is_stable=True)
    counts = jnp.bincount(cluster_num.astype(jnp.int32), length=NUM_K)
    starts = jnp.concatenate(
        [jnp.zeros((1,), jnp.int32),
         jnp.cumsum(counts, dtype=jnp.int32)]).astype(jnp.int32)
    starts_p = jnp.pad(starts, (0, STARTS_PAD - (NUM_K + 1)),
                       constant_values=M)
    rows_p = jnp.pad(order, (0, UPD_PAD - M))
    scol_p = jnp.pad(scol, (0, UPD_PAD - M))

    mesh = plsc.VectorSubcoreMesh(core_axis_name="c", subcore_axis_name="s",
                                  num_cores=2, num_subcores=16)
    run = pl.kernel(
        _sc_body,
        out_type=jax.ShapeDtypeStruct((FEAT, TOTAL), jnp.float32),
        mesh=mesh,
        scratch_types=[
            pltpu.VMEM((STARTS_LEN,), jnp.int32),
            pltpu.VMEM((3, IDXBUF), jnp.int32),
            pltpu.VMEM((3, SCOL_LEN), jnp.int32),
            pltpu.VMEM((3, IDXBUF, FEAT), jnp.float32),
            pltpu.VMEM((3, FEAT, POOL_PER), jnp.float32),
            pltpu.SemaphoreType.DMA((3,)),
            pltpu.SemaphoreType.DMA((3,)),
            pltpu.SemaphoreType.DMA((3,)),
            pltpu.SemaphoreType.DMA((3,)),
        ],
        compiler_params=pltpu.CompilerParams(use_tc_tiling_on_sc=True,
                                             needs_layout_passes=False),
    )
    return run(activation, rows_p, scol_p, starts_p, concept_pool)


# speedup vs baseline: 16.4424x; 1.0477x over previous
import jax
import jax.numpy as jnp
from jax import lax
from jax.experimental import pallas as pl
from jax.experimental.pallas import tpu as pltpu
from jax.experimental.pallas import tpu_sc as plsc

FEAT = 128
NUM_K = 1024
POOL_PER = 256
TOTAL = NUM_K * POOL_PER
M = 16384

NUM_TILES = 32
CPT = NUM_K // NUM_TILES
CAP = 33
IDXBUF = 40
UPD_PAD = M + 64


def _sc_body(act_hbm, rows_hbm, scol_hbm, pool_hbm, out_hbm,
             scol_full, rowid_v, rows_v, blocks_v, lsem, ssem, isem, gsem):
    wid = lax.axis_index("c") * 16 + lax.axis_index("s")
    c0 = wid * CPT

    pltpu.sync_copy(scol_hbm, scol_full)

    def lower_bound(v):
        def bs(_, lohi):
            lo, hi = lohi
            mid = (lo + hi) // 2
            x = scol_full[pl.ds(mid, 16)][0]
            go = jnp.logical_and(lo < hi, x < v)
            shrink = jnp.logical_and(lo < hi, x >= v)
            return (jnp.where(go, mid + 1, lo), jnp.where(shrink, mid, hi))

        lo, _ = lax.fori_loop(0, 15, bs, (jnp.int32(0), jnp.int32(M)))
        return lo

    def col_window(j):
        return pl.ds((c0 + j) * POOL_PER, POOL_PER)

    def start_idx(s, b):
        a = (s // 8) * 8
        pltpu.async_copy(rows_hbm.at[pl.ds(a, IDXBUF)], rowid_v.at[b],
                         isem.at[b])

    def wait_idx(b):
        pltpu.make_async_copy(rows_hbm.at[pl.ds(0, IDXBUF)], rowid_v.at[b],
                              isem.at[b]).wait()

    def issue_gather(b):
        pltpu.async_copy(act_hbm.at[rowid_v.at[b]], rows_v.at[b], gsem.at[b])

    def wait_gather(b):
        pltpu.make_async_copy(act_hbm.at[rowid_v.at[b]], rows_v.at[b],
                              gsem.at[b]).wait()

    def start_load(j, b):
        pltpu.async_copy(pool_hbm.at[:, col_window(j)], blocks_v.at[b],
                         lsem.at[b])

    def wait_load(j, b):
        pltpu.make_async_copy(pool_hbm.at[:, col_window(j)], blocks_v.at[b],
                              lsem.at[b]).wait()

    def start_store(j, b):
        pltpu.async_copy(blocks_v.at[b], out_hbm.at[:, col_window(j)],
                         ssem.at[b])

    def wait_store(b):
        pltpu.make_async_copy(blocks_v.at[b], out_hbm.at[:, col_window(0)],
                              ssem.at[b]).wait()

    def apply_range(lo, hi, a, b, col0):
        def apply(p, _):
            q = p - a
            o = scol_full[pl.ds(p, 16)][0] - col0
            cidx = jnp.full((16,), o, dtype=jnp.int32)
            for fv in range(FEAT // 16):
                vals = rows_v[b, q, pl.ds(fv * 16, 16)]
                ridx = lax.iota(jnp.int32, 16) + fv * 16
                plsc.store_scatter(blocks_v.at[b], [ridx, cidx], vals)
            return 0

        lax.fori_loop(lo, hi, apply, 0)

    def apply_cluster(j, b, s, e):
        col0 = (c0 + j) * POOL_PER
        apply_range(s, jnp.minimum(s + CAP, e), (s // 8) * 8, b, col0)
        nch = (e - s + (CAP - 1)) // CAP

        def rare(k, _):
            base = s + k * CAP
            a = (base // 8) * 8
            pltpu.sync_copy(rows_hbm.at[pl.ds(a, IDXBUF)], rowid_v.at[b])
            pltpu.async_copy(act_hbm.at[rowid_v.at[b]], rows_v.at[b],
                             gsem.at[b]).wait()
            apply_range(base, jnp.minimum(base + CAP, e), a, b, col0)
            return 0

        lax.fori_loop(1, nch, rare, 0)

    def step(j, r, s_j, s_j1):
        nb = (r + 1) % 3
        pb = (r + 2) % 3

        @pl.when(j >= 2)
        def _():
            wait_store(nb)

        start_load(j + 1, nb)
        s_j2 = lower_bound((c0 + j + 2) * POOL_PER)
        start_idx(s_j2, pb)
        wait_idx(nb)
        issue_gather(nb)
        wait_load(j, r)
        wait_gather(r)
        apply_cluster(j, r, s_j, s_j1)
        start_store(j, r)
        return s_j1, s_j2

    s0 = lower_bound(c0 * POOL_PER)
    s1 = lower_bound((c0 + 1) * POOL_PER)
    start_idx(s0, 0)
    wait_idx(0)
    issue_gather(0)
    start_idx(s1, 1)
    start_load(0, 0)

    def loop(i, carry):
        a, b = carry
        for r in range(3):
            a, b = step(3 * i + r, r, a, b)
        return a, b

    s30, s31 = lax.fori_loop(0, 10, loop, (s0, s1))

    s32 = lower_bound((c0 + 32) * POOL_PER)
    wait_store(1)
    start_load(31, 1)
    wait_idx(1)
    issue_gather(1)
    wait_load(30, 0)
    wait_gather(0)
    apply_cluster(30, 0, s30, s31)
    start_store(30, 0)

    wait_load(31, 1)
    wait_gather(1)
    apply_cluster(31, 1, s31, s32)
    start_store(31, 1)

    wait_store(0)
    wait_store(1)
    wait_store(2)


def kernel(activation, cluster_num, rand_offsets, concept_pool):
    idx = (cluster_num.astype(jnp.int32) * POOL_PER
           + rand_offsets.astype(jnp.int32))
    scol, order = lax.sort_key_val(idx, jnp.arange(M, dtype=jnp.int32),
                                   is_stable=True)
    rows_p = jnp.pad(order, (0, UPD_PAD - M))
    scol_p = jnp.pad(scol, (0, UPD_PAD - M))

    mesh = plsc.VectorSubcoreMesh(core_axis_name="c", subcore_axis_name="s",
                                  num_cores=2, num_subcores=16)
    run = pl.kernel(
        _sc_body,
        out_type=jax.ShapeDtypeStruct((FEAT, TOTAL), jnp.float32),
        mesh=mesh,
        scratch_types=[
            pltpu.VMEM((UPD_PAD,), jnp.int32),
            pltpu.VMEM((3, IDXBUF), jnp.int32),
            pltpu.VMEM((3, IDXBUF, FEAT), jnp.float32),
            pltpu.VMEM((3, FEAT, POOL_PER), jnp.float32),
            pltpu.SemaphoreType.DMA((3,)),
            pltpu.SemaphoreType.DMA((3,)),
            pltpu.SemaphoreType.DMA((3,)),
            pltpu.SemaphoreType.DMA((3,)),
        ],
        compiler_params=pltpu.CompilerParams(use_tc_tiling_on_sc=True,
                                             needs_layout_passes=False),
    )
    return run(activation, rows_p, scol_p, concept_pool)


# speedup vs baseline: 16.5799x; 1.0084x over previous
import jax
import jax.numpy as jnp
from jax import lax
from jax.experimental import pallas as pl
from jax.experimental.pallas import tpu as pltpu
from jax.experimental.pallas import tpu_sc as plsc

FEAT = 128
NUM_K = 1024
POOL_PER = 256
TOTAL = NUM_K * POOL_PER
M = 16384

NUM_TILES = 32
CPT = NUM_K // NUM_TILES
CAP = 33
IDXBUF = 40
UPD_PAD = M + 64


def _sc_body(act_hbm, rows_hbm, scol_hbm, pool_hbm, out_hbm,
             scol_full, rowid_v, rows_v, blocks_v, lsem, ssem, isem, gsem):
    wid = lax.axis_index("c") * 16 + lax.axis_index("s")
    c0 = wid * CPT

    def lower_bound(v):
        def bs(_, lohi):
            lo, hi = lohi
            mid = (lo + hi) // 2
            x = scol_full[pl.ds(mid, 16)][0]
            go = jnp.logical_and(lo < hi, x < v)
            shrink = jnp.logical_and(lo < hi, x >= v)
            return (jnp.where(go, mid + 1, lo), jnp.where(shrink, mid, hi))

        lo, _ = lax.fori_loop(0, 15, bs, (jnp.int32(0), jnp.int32(M)))
        return lo

    def col_window(j):
        return pl.ds((c0 + j) * POOL_PER, POOL_PER)

    def start_idx(s, b):
        a = (s // 8) * 8
        pltpu.async_copy(rows_hbm.at[pl.ds(a, IDXBUF)], rowid_v.at[b],
                         isem.at[b])

    def wait_idx(b):
        pltpu.make_async_copy(rows_hbm.at[pl.ds(0, IDXBUF)], rowid_v.at[b],
                              isem.at[b]).wait()

    def issue_gather(b):
        pltpu.async_copy(act_hbm.at[rowid_v.at[b]], rows_v.at[b], gsem.at[b])

    def wait_gather(b):
        pltpu.make_async_copy(act_hbm.at[rowid_v.at[b]], rows_v.at[b],
                              gsem.at[b]).wait()

    def start_load(j, b):
        pltpu.async_copy(pool_hbm.at[:, col_window(j)], blocks_v.at[b],
                         lsem.at[b])

    def wait_load(j, b):
        pltpu.make_async_copy(pool_hbm.at[:, col_window(j)], blocks_v.at[b],
                              lsem.at[b]).wait()

    def start_store(j, b):
        pltpu.async_copy(blocks_v.at[b], out_hbm.at[:, col_window(j)],
                         ssem.at[b])

    def wait_store(b):
        pltpu.make_async_copy(blocks_v.at[b], out_hbm.at[:, col_window(0)],
                              ssem.at[b]).wait()

    def apply_range(lo, hi, a, b, col0):
        def apply(p, _):
            q = p - a
            o = scol_full[pl.ds(p, 16)][0] - col0
            cidx = jnp.full((16,), o, dtype=jnp.int32)
            for fv in range(FEAT // 16):
                vals = rows_v[b, q, pl.ds(fv * 16, 16)]
                ridx = lax.iota(jnp.int32, 16) + fv * 16
                plsc.store_scatter(blocks_v.at[b], [ridx, cidx], vals)
            return 0

        lax.fori_loop(lo, hi, apply, 0)

    def apply_cluster(j, b, s, e):
        col0 = (c0 + j) * POOL_PER
        apply_range(s, jnp.minimum(s + CAP, e), (s // 8) * 8, b, col0)
        nch = (e - s + (CAP - 1)) // CAP

        def rare(k, _):
            base = s + k * CAP
            a = (base // 8) * 8
            pltpu.sync_copy(rows_hbm.at[pl.ds(a, IDXBUF)], rowid_v.at[b])
            pltpu.async_copy(act_hbm.at[rowid_v.at[b]], rows_v.at[b],
                             gsem.at[b]).wait()
            apply_range(base, jnp.minimum(base + CAP, e), a, b, col0)
            return 0

        lax.fori_loop(1, nch, rare, 0)

    def step(j, r, s_j, s_j1):
        nb = (r + 1) % 3
        pb = (r + 2) % 3

        @pl.when(j >= 2)
        def _():
            wait_store(nb)

        start_load(j + 1, nb)
        s_j2 = lower_bound((c0 + j + 2) * POOL_PER)
        start_idx(s_j2, pb)
        wait_idx(nb)
        issue_gather(nb)
        wait_load(j, r)
        wait_gather(r)
        apply_cluster(j, r, s_j, s_j1)
        start_store(j, r)
        return s_j1, s_j2

    start_load(0, 0)
    pltpu.sync_copy(scol_hbm, scol_full)

    s0 = lower_bound(c0 * POOL_PER)
    s1 = lower_bound((c0 + 1) * POOL_PER)
    start_idx(s0, 0)
    wait_idx(0)
    issue_gather(0)
    start_idx(s1, 1)

    def loop(i, carry):
        a, b = carry
        for r in range(3):
            a, b = step(3 * i + r, r, a, b)
        return a, b

    s30, s31 = lax.fori_loop(0, 10, loop, (s0, s1))

    s32 = lower_bound((c0 + 32) * POOL_PER)
    wait_store(1)
    start_load(31, 1)
    wait_idx(1)
    issue_gather(1)
    wait_load(30, 0)
    wait_gather(0)
    apply_cluster(30, 0, s30, s31)
    start_store(30, 0)

    wait_load(31, 1)
    wait_gather(1)
    apply_cluster(31, 1, s31, s32)
    start_store(31, 1)

    wait_store(0)
    wait_store(1)
    wait_store(2)


def kernel(activation, cluster_num, rand_offsets, concept_pool):
    idx = (cluster_num.astype(jnp.int32) * POOL_PER
           + rand_offsets.astype(jnp.int32))
    scol, order = lax.sort_key_val(idx, jnp.arange(M, dtype=jnp.int32),
                                   is_stable=True)
    rows_p = jnp.pad(order, (0, UPD_PAD - M))
    scol_p = jnp.pad(scol, (0, UPD_PAD - M))

    mesh = plsc.VectorSubcoreMesh(core_axis_name="c", subcore_axis_name="s",
                                  num_cores=2, num_subcores=16)
    run = pl.kernel(
        _sc_body,
        out_type=jax.ShapeDtypeStruct((FEAT, TOTAL), jnp.float32),
        mesh=mesh,
        scratch_types=[
            pltpu.VMEM((UPD_PAD,), jnp.int32),
            pltpu.VMEM((3, IDXBUF), jnp.int32),
            pltpu.VMEM((3, IDXBUF, FEAT), jnp.float32),
            pltpu.VMEM((3, FEAT, POOL_PER), jnp.float32),
            pltpu.SemaphoreType.DMA((3,)),
            pltpu.SemaphoreType.DMA((3,)),
            pltpu.SemaphoreType.DMA((3,)),
            pltpu.SemaphoreType.DMA((3,)),
        ],
        compiler_params=pltpu.CompilerParams(use_tc_tiling_on_sc=True,
                                             needs_layout_passes=False),
    )
    return run(activation, rows_p, scol_p, concept_pool)
